# baseline (speedup 1.0000x reference)
import functools

import jax
import jax.numpy as jnp
from jax import lax
from jax.experimental import pallas as pl
from jax.experimental.pallas import tpu as pltpu
from jax.experimental.pallas import tpu_sc as plsc

_NW = 32
_CB = 16


def _make_sc_gather(n_rows, nb, fout):
    mesh = plsc.VectorSubcoreMesh(core_axis_name="c", subcore_axis_name="s")
    e_total = nb * 128
    nbw = nb // _NW
    nchunks = nbw // _CB

    @functools.partial(
        pl.kernel,
        mesh=mesh,
        out_type=jax.ShapeDtypeStruct((8, e_total), jnp.float32),
        compiler_params=pltpu.CompilerParams(
            needs_layout_passes=False, use_tc_tiling_on_sc=False),
        scratch_types=[
            pltpu.VMEM((_CB, 128), jnp.int32),
            pltpu.VMEM((_CB * 128, 16), jnp.float32),
            pltpu.VMEM((16 * _CB * 128,), jnp.float32),
            pltpu.SemaphoreType.DMA,
        ],
    )
    def gather_k(table_hbm, idx_hbm, out_hbm, idx_v, rec_v, f1_v, sem):
        wid = lax.axis_index("s") * 2 + lax.axis_index("c")
        start = wid * nbw
        cw = _CB * 128
        iota = lax.iota(jnp.int32, 16)
        fbase = iota * cw

        def chunk_body(c, carry):
            bstart = start + c * _CB
            pltpu.sync_copy(idx_hbm.at[pl.ds(bstart, _CB)], idx_v)
            cps = []
            for i in range(_CB):
                cps.append(pltpu.async_copy(
                    table_hbm.at[idx_v.at[i]],
                    rec_v.at[pl.ds(i * 128, 128), :], sem))
            for cp in cps:
                cp.wait()

            def rec_body(b, carry2):
                for j in range(16):
                    r = b * 16 + j
                    vec = rec_v[r, :]
                    plsc.store_scatter(f1_v, [fbase + r], vec)
                return carry2

            lax.fori_loop(0, _CB * 8, rec_body, 0)
            ops = []
            for f in range(8):
                ops.append(pltpu.async_copy(
                    f1_v.at[pl.ds(f * cw, cw)],
                    out_hbm.at[f, pl.ds(bstart * 128, cw)], sem))
            for op in ops:
                op.wait()
            return carry

        lax.fori_loop(0, nchunks, chunk_body, 0)

    return gather_k


def _mlp2(l0, l1, x):
    y = jax.nn.relu(x @ l0["W"].T + l0["b"])
    return y @ l1["W"].T + l1["b"]


def _pick_block(e):
    for cand in (2560, 1280, 640, 512, 256, 128, 64, 32, 16, 8):
        if e % cand == 0:
            return cand
    return e


def _edge_mlp_kernel(w1u_ref, w1p_ref, b1_ref, w2_ref, b2_ref, u_ref, pd_ref,
                     o_ref):
    z = jnp.dot(w1u_ref[...], u_ref[...], preferred_element_type=jnp.float32)
    z = z + jnp.dot(w1p_ref[...], pd_ref[...],
                    preferred_element_type=jnp.float32)
    z = jax.nn.relu(z + b1_ref[...])
    o_ref[...] = (jnp.dot(w2_ref[...], z, preferred_element_type=jnp.float32)
                  + b2_ref[...])


@functools.partial(jax.jit, static_argnames=())
def _edge_mlp(w1u, w1p, b1, w2, b2, u_t, pd_t):
    e = u_t.shape[1]
    eb = _pick_block(e)
    grid = (e // eb,)
    full = lambda i: (0, 0)
    blk = lambda i: (0, i)
    return pl.pallas_call(
        _edge_mlp_kernel,
        grid=grid,
        in_specs=[
            pl.BlockSpec(w1u.shape, full),
            pl.BlockSpec(w1p.shape, full),
            pl.BlockSpec(b1.shape, full),
            pl.BlockSpec(w2.shape, full),
            pl.BlockSpec(b2.shape, full),
            pl.BlockSpec((u_t.shape[0], eb), blk),
            pl.BlockSpec((pd_t.shape[0], eb), blk),
        ],
        out_specs=pl.BlockSpec((3, eb), blk),
        out_shape=jax.ShapeDtypeStruct((3, e), jnp.float32),
    )(w1u, w1p, b1, w2, b2, u_t, pd_t)


@functools.lru_cache(maxsize=None)
def _cached_gather(n_nodes, nb, fout):
    return _make_sc_gather(n_nodes, nb, fout)


_NH = 25088
_SCW = 2048


def _make_sc_scatter_max(e_total):
    mesh = plsc.VectorSubcoreMesh(core_axis_name="c", subcore_axis_name="s")
    ew = e_total // _NW
    nchunks = ew // _SCW
    tsz = _NH * 3 + 16

    @functools.partial(
        pl.kernel,
        mesh=mesh,
        out_type=jax.ShapeDtypeStruct((_NW, 2, _NH * 3), jnp.float32),
        compiler_params=pltpu.CompilerParams(
            needs_layout_passes=False, use_tc_tiling_on_sc=False),
        scratch_types=[
            pltpu.VMEM((tsz,), jnp.float32),
            pltpu.VMEM((3 * _SCW,), jnp.float32),
            pltpu.VMEM((_SCW,), jnp.int32),
            pltpu.VMEM((16,), jnp.int32),
            pltpu.VMEM((16,), jnp.float32),
            pltpu.SemaphoreType.DMA,
        ],
    )
    def scatter_k(msg_hbm, dst_hbm, ninf_hbm, out_hbm,
                  tab_v, m_v, d_v, s16, c16, sem):
        wid = lax.axis_index("s") * 2 + lax.axis_index("c")
        ebase = wid * ew
        iota = lax.iota(jnp.int32, 16)
        ninf = jnp.full((16,), -jnp.inf, jnp.float32)
        trash = _NH * 3 + iota
        idx_m = [jnp.maximum(iota - k, 0) for k in (1, 2, 4, 8)]
        idx_p1 = jnp.minimum(iota + 1, 15)

        for p in range(2):
            base = p * _NH
            pltpu.sync_copy(ninf_hbm, tab_v)

            def chunk_body(ch, carry):
                estart = ebase + ch * _SCW
                pltpu.sync_copy(dst_hbm.at[pl.ds(estart, _SCW)], d_v)
                cps = [pltpu.async_copy(
                    msg_hbm.at[c, pl.ds(estart, _SCW)],
                    m_v.at[pl.ds(c * _SCW, _SCW)], sem) for c in range(3)]
                for cp in cps:
                    cp.wait()

                def vreg_body(b, carry2):
                    dvec = d_v[pl.ds(b * 16, 16)]
                    lvec = dvec - base
                    cnt, _ = plsc.scan_count(dvec)
                    has_dup = jnp.max(cnt) > 1

                    def fast(_):
                        valid = (lvec >= 0) & (lvec < _NH)
                        cl3 = jnp.clip(lvec, 0, _NH - 1) * 3
                        for c in range(3):
                            val = m_v[pl.ds(b * 16 + c * _SCW, 16)]
                            cur = plsc.load_gather(tab_v, [cl3 + c])
                            comb = jnp.maximum(
                                val, jnp.where(valid, cur, ninf))
                            tidx = jnp.where(valid, cl3 + c, trash)
                            plsc.store_scatter(tab_v, [tidx], comb)
                        return 0

                    def slow(_):
                        skey, perm = plsc.sort_key_val(lvec, iota)
                        s16[...] = skey
                        svalid = (skey >= 0) & (skey < _NH)
                        masks = []
                        for k, im in zip((1, 2, 4, 8), idx_m):
                            sh = plsc.load_gather(s16, [im])
                            masks.append((skey == sh) & (iota >= k))
                        shp1 = plsc.load_gather(s16, [idx_p1])
                        lastm = (skey != shp1) | (iota == 15)
                        storem = lastm & svalid
                        cl3 = jnp.clip(skey, 0, _NH - 1) * 3
                        for c in range(3):
                            val = plsc.load_gather(
                                m_v, [c * _SCW + b * 16 + perm])
                            cur = plsc.load_gather(tab_v, [cl3 + c])
                            comb = jnp.maximum(
                                val, jnp.where(svalid, cur, ninf))
                            for mk, im in zip(masks, idx_m):
                                c16[...] = comb
                                shc = plsc.load_gather(c16, [im])
                                comb = jnp.where(
                                    mk, jnp.maximum(comb, shc), comb)
                            tidx = jnp.where(storem, cl3 + c, trash)
                            plsc.store_scatter(tab_v, [tidx], comb)
                        return 0

                    lax.cond(has_dup, slow, fast, 0)
                    return carry2

                lax.fori_loop(0, _SCW // 16, vreg_body, 0)
                return carry

            lax.fori_loop(0, nchunks, chunk_body, 0)
            pltpu.sync_copy(tab_v.at[pl.ds(0, _NH * 3)], out_hbm.at[wid, p])

    return scatter_k


@functools.lru_cache(maxsize=None)
def _cached_scatter(e_total):
    return _make_sc_scatter_max(e_total)


def kernel(x, params, edge_index, batch):
    n = x.shape[0]
    g = 64
    src = edge_index[0]
    dst = edge_index[1]
    e = src.shape[0]
    unit = 128 * _NW * _CB
    e_pad = -(-e // unit) * unit
    nb = e_pad // 128
    pos = x
    src_p = jnp.concatenate([src, jnp.full((e_pad - e,), n, jnp.int32)])
    dst_p = jnp.concatenate([dst, jnp.full((e_pad - e,), n, jnp.int32)])
    src2d = src_p.reshape(nb, 128)
    dst2d = dst_p.reshape(nb, 128)
    zrow = jnp.zeros((1, 16), jnp.float32)
    zpad = jnp.zeros((n, 10), jnp.float32)
    pos16 = jnp.concatenate(
        [jnp.concatenate([pos, jnp.zeros((n, 13), jnp.float32)], axis=1),
         zrow], axis=0)
    pd_t = _cached_gather(n + 1, nb, 3)(pos16, dst2d)

    h = x
    for ci in range(3):
        p = params["conv%d" % (ci + 1)]
        hn = _mlp2(p["h"][0], p["h"][1], h)
        tab16 = jnp.concatenate(
            [jnp.concatenate([hn, pos, zpad], axis=1), zrow], axis=0)
        u_t = _cached_gather(n + 1, nb, 6)(tab16, src2d)

        wf1 = p["f"][0]["W"]
        b1 = p["f"][0]["b"][:, None]
        w1u = jnp.pad(wf1, ((0, 0), (0, 2)))
        w1p = jnp.pad(-wf1[:, 3:], ((0, 0), (0, 5)))
        w2 = p["f"][1]["W"]
        b2 = p["f"][1]["b"][:, None]

        msg_t = _edge_mlp(w1u, w1p, b1, w2, b2, u_t, pd_t)
        ninf_init = jnp.full((_NH * 3 + 16,), -jnp.inf, jnp.float32)
        part = _cached_scatter(e_pad)(msg_t, dst_p, ninf_init)
        agg = jnp.max(part, axis=0).reshape(2 * _NH, 3)[:n]
        agg = jnp.where(jnp.isfinite(agg), agg, 0.0)
        h = _mlp2(p["g"][0], p["g"][1], agg)
        if ci < 2:
            h = jax.nn.relu(h)

    ones = jnp.ones((n,), dtype=jnp.float32)
    counts = jax.ops.segment_sum(ones, batch, num_segments=g)
    sums = jax.ops.segment_sum(h, batch, num_segments=g)
    pooled = sums / jnp.maximum(counts, 1.0)[:, None]
    logits = pooled @ params["linear"]["W"].T + params["linear"]["b"]
    return jax.nn.log_softmax(logits, axis=1)


# speedup vs baseline: 1.0264x; 1.0264x over previous
import functools

import jax
import jax.numpy as jnp
from jax import lax
from jax.experimental import pallas as pl
from jax.experimental.pallas import tpu as pltpu
from jax.experimental.pallas import tpu_sc as plsc

_NW = 32
_CB = 16


def _make_sc_gather(n_rows, nb, fout):
    mesh = plsc.VectorSubcoreMesh(core_axis_name="c", subcore_axis_name="s")
    e_total = nb * 128
    nbw = nb // _NW
    nchunks = nbw // _CB

    @functools.partial(
        pl.kernel,
        mesh=mesh,
        out_type=jax.ShapeDtypeStruct((8, e_total), jnp.float32),
        compiler_params=pltpu.CompilerParams(
            needs_layout_passes=False, use_tc_tiling_on_sc=False),
        scratch_types=[
            pltpu.VMEM((_CB, 128), jnp.int32),
            pltpu.VMEM((_CB * 128, 16), jnp.float32),
            pltpu.VMEM((16 * _CB * 128,), jnp.float32),
            pltpu.SemaphoreType.DMA,
        ],
    )
    def gather_k(table_hbm, idx_hbm, out_hbm, idx_v, rec_v, f1_v, sem):
        wid = lax.axis_index("s") * 2 + lax.axis_index("c")
        start = wid * nbw
        cw = _CB * 128
        iota = lax.iota(jnp.int32, 16)
        fbase = iota * cw

        def chunk_body(c, carry):
            bstart = start + c * _CB
            pltpu.sync_copy(idx_hbm.at[pl.ds(bstart, _CB)], idx_v)
            cps = []
            for i in range(_CB):
                cps.append(pltpu.async_copy(
                    table_hbm.at[idx_v.at[i]],
                    rec_v.at[pl.ds(i * 128, 128), :], sem))
            for cp in cps:
                cp.wait()

            def rec_body(b, carry2):
                for j in range(16):
                    r = b * 16 + j
                    vec = rec_v[r, :]
                    plsc.store_scatter(f1_v, [fbase + r], vec)
                return carry2

            lax.fori_loop(0, _CB * 8, rec_body, 0)
            ops = []
            for f in range(8):
                ops.append(pltpu.async_copy(
                    f1_v.at[pl.ds(f * cw, cw)],
                    out_hbm.at[f, pl.ds(bstart * 128, cw)], sem))
            for op in ops:
                op.wait()
            return carry

        lax.fori_loop(0, nchunks, chunk_body, 0)

    return gather_k


def _mlp2(l0, l1, x):
    y = jax.nn.relu(x @ l0["W"].T + l0["b"])
    return y @ l1["W"].T + l1["b"]


def _pick_block(e):
    for cand in (2560, 1280, 640, 512, 256, 128, 64, 32, 16, 8):
        if e % cand == 0:
            return cand
    return e


def _edge_mlp_kernel(w1u_ref, w1p_ref, b1_ref, w2_ref, b2_ref, u_ref, pd_ref,
                     o_ref):
    z = jnp.dot(w1u_ref[...], u_ref[...], preferred_element_type=jnp.float32)
    z = z + jnp.dot(w1p_ref[...], pd_ref[...],
                    preferred_element_type=jnp.float32)
    z = jax.nn.relu(z + b1_ref[...])
    o_ref[...] = (jnp.dot(w2_ref[...], z, preferred_element_type=jnp.float32)
                  + b2_ref[...])


@functools.partial(jax.jit, static_argnames=())
def _edge_mlp(w1u, w1p, b1, w2, b2, u_t, pd_t):
    e = u_t.shape[1]
    eb = _pick_block(e)
    grid = (e // eb,)
    full = lambda i: (0, 0)
    blk = lambda i: (0, i)
    return pl.pallas_call(
        _edge_mlp_kernel,
        grid=grid,
        in_specs=[
            pl.BlockSpec(w1u.shape, full),
            pl.BlockSpec(w1p.shape, full),
            pl.BlockSpec(b1.shape, full),
            pl.BlockSpec(w2.shape, full),
            pl.BlockSpec(b2.shape, full),
            pl.BlockSpec((u_t.shape[0], eb), blk),
            pl.BlockSpec((pd_t.shape[0], eb), blk),
        ],
        out_specs=pl.BlockSpec((3, eb), blk),
        out_shape=jax.ShapeDtypeStruct((3, e), jnp.float32),
    )(w1u, w1p, b1, w2, b2, u_t, pd_t)


@functools.lru_cache(maxsize=None)
def _cached_gather(n_nodes, nb, fout):
    return _make_sc_gather(n_nodes, nb, fout)


_NH = 25088
_SCW = 2048


def _make_sc_scatter_max(e_total):
    mesh = plsc.VectorSubcoreMesh(core_axis_name="c", subcore_axis_name="s")
    ew = e_total // _NW
    nchunks = ew // _SCW
    tsz = _NH * 3 + 16

    @functools.partial(
        pl.kernel,
        mesh=mesh,
        out_type=jax.ShapeDtypeStruct((_NW, 2, _NH * 3), jnp.float32),
        compiler_params=pltpu.CompilerParams(
            needs_layout_passes=False, use_tc_tiling_on_sc=False),
        scratch_types=[
            pltpu.VMEM((tsz,), jnp.float32),
            pltpu.VMEM((3 * _SCW,), jnp.float32),
            pltpu.VMEM((_SCW,), jnp.int32),
            pltpu.VMEM((16,), jnp.int32),
            pltpu.VMEM((16,), jnp.float32),
            pltpu.SemaphoreType.DMA,
        ],
    )
    def scatter_k(msg_hbm, dst_hbm, ninf_hbm, out_hbm,
                  tab_v, m_v, d_v, s16, c16, sem):
        wid = lax.axis_index("s") * 2 + lax.axis_index("c")
        ebase = wid * ew
        iota = lax.iota(jnp.int32, 16)
        ninf = jnp.full((16,), -jnp.inf, jnp.float32)
        trash = _NH * 3 + iota
        idx_m = [jnp.maximum(iota - k, 0) for k in (1, 2, 4, 8)]
        idx_p1 = jnp.minimum(iota + 1, 15)

        for p in range(2):
            base = p * _NH
            pltpu.sync_copy(ninf_hbm, tab_v)

            def chunk_body(ch, carry):
                estart = ebase + ch * _SCW
                pltpu.sync_copy(dst_hbm.at[pl.ds(estart, _SCW)], d_v)
                cps = [pltpu.async_copy(
                    msg_hbm.at[c, pl.ds(estart, _SCW)],
                    m_v.at[pl.ds(c * _SCW, _SCW)], sem) for c in range(3)]
                for cp in cps:
                    cp.wait()

                def vreg_body(b, carry2):
                    dvec = d_v[pl.ds(b * 16, 16)]
                    lvec = dvec - base
                    cnt, _ = plsc.scan_count(dvec)
                    has_dup = jnp.max(cnt) > 1

                    def fast(_):
                        valid = (lvec >= 0) & (lvec < _NH)
                        cl3 = jnp.clip(lvec, 0, _NH - 1) * 3
                        for c in range(3):
                            val = m_v[pl.ds(b * 16 + c * _SCW, 16)]
                            cur = plsc.load_gather(tab_v, [cl3 + c])
                            comb = jnp.maximum(
                                val, jnp.where(valid, cur, ninf))
                            tidx = jnp.where(valid, cl3 + c, trash)
                            plsc.store_scatter(tab_v, [tidx], comb)
                        return 0

                    def slow(_):
                        skey, perm = plsc.sort_key_val(lvec, iota)
                        s16[...] = skey
                        svalid = (skey >= 0) & (skey < _NH)
                        masks = []
                        for k, im in zip((1, 2, 4, 8), idx_m):
                            sh = plsc.load_gather(s16, [im])
                            masks.append((skey == sh) & (iota >= k))
                        shp1 = plsc.load_gather(s16, [idx_p1])
                        lastm = (skey != shp1) | (iota == 15)
                        storem = lastm & svalid
                        cl3 = jnp.clip(skey, 0, _NH - 1) * 3
                        for c in range(3):
                            val = plsc.load_gather(
                                m_v, [c * _SCW + b * 16 + perm])
                            cur = plsc.load_gather(tab_v, [cl3 + c])
                            comb = jnp.maximum(
                                val, jnp.where(svalid, cur, ninf))
                            for mk, im in zip(masks, idx_m):
                                c16[...] = comb
                                shc = plsc.load_gather(c16, [im])
                                comb = jnp.where(
                                    mk, jnp.maximum(comb, shc), comb)
                            tidx = jnp.where(storem, cl3 + c, trash)
                            plsc.store_scatter(tab_v, [tidx], comb)
                        return 0

                    lax.cond(has_dup, slow, fast, 0)
                    return carry2

                lax.fori_loop(0, _SCW // 16, vreg_body, 0)
                return carry

            lax.fori_loop(0, nchunks, chunk_body, 0)
            pltpu.sync_copy(tab_v.at[pl.ds(0, _NH * 3)], out_hbm.at[wid, p])

    return scatter_k


@functools.lru_cache(maxsize=None)
def _cached_scatter(e_total):
    return _make_sc_scatter_max(e_total)


def kernel(x, params, edge_index, batch):
    n = x.shape[0]
    g = 64
    src = edge_index[0]
    dst = edge_index[1]
    e = src.shape[0]
    unit = 128 * _NW * _CB
    e_pad = -(-e // unit) * unit
    nb = e_pad // 128
    pos = x
    src_p = jnp.concatenate([src, jnp.full((e_pad - e,), n, jnp.int32)])
    dst_p = jnp.concatenate([dst, jnp.full((e_pad - e,), n, jnp.int32)])
    src2d = src_p.reshape(nb, 128)
    dst2d = dst_p.reshape(nb, 128)
    zrow = jnp.zeros((1, 16), jnp.float32)
    zpad = jnp.zeros((n, 10), jnp.float32)
    pos16 = jnp.concatenate(
        [jnp.concatenate([pos, jnp.zeros((n, 13), jnp.float32)], axis=1),
         zrow], axis=0)
    pd_t = _cached_gather(n + 1, nb, 3)(pos16, dst2d)

    h = x
    for ci in range(3):
        p = params["conv%d" % (ci + 1)]
        hn = _mlp2(p["h"][0], p["h"][1], h)
        tab16 = jnp.concatenate(
            [jnp.concatenate([hn, pos, zpad], axis=1), zrow], axis=0)
        u_t = _cached_gather(n + 1, nb, 6)(tab16, src2d)

        wf1 = p["f"][0]["W"]
        b1 = p["f"][0]["b"][:, None]
        w1u = jnp.pad(wf1, ((0, 0), (0, 2)))
        w1p = jnp.pad(-wf1[:, 3:], ((0, 0), (0, 5)))
        w2 = p["f"][1]["W"]
        b2 = p["f"][1]["b"][:, None]

        msg_t = _edge_mlp(w1u, w1p, b1, w2, b2, u_t, pd_t)
        ninf_init = jnp.full((_NH * 3 + 16,), -jnp.inf, jnp.float32)
        part = _cached_scatter(e_pad)(msg_t, dst_p, ninf_init)
        agg = jnp.max(part, axis=0).reshape(2 * _NH, 3)[:n]
        agg = jnp.where(jnp.isfinite(agg), agg, 0.0)
        h = _mlp2(p["g"][0], p["g"][1], agg)
        if ci < 2:
            h = jax.nn.relu(h)

    onehot = (batch[None, :] == jnp.arange(g, dtype=batch.dtype)[:, None]
              ).astype(jnp.float32)
    counts = jnp.sum(onehot, axis=1)
    sums = onehot @ h
    pooled = sums / jnp.maximum(counts, 1.0)[:, None]
    logits = pooled @ params["linear"]["W"].T + params["linear"]["b"]
    return jax.nn.log_softmax(logits, axis=1)


# speedup vs baseline: 1.1903x; 1.1597x over previous
import functools

import jax
import jax.numpy as jnp
from jax import lax
from jax.experimental import pallas as pl
from jax.experimental.pallas import tpu as pltpu
from jax.experimental.pallas import tpu_sc as plsc

_NW = 32
_CB = 16


def _make_sc_gather(n_rows, nb, fout):
    mesh = plsc.VectorSubcoreMesh(core_axis_name="c", subcore_axis_name="s")
    e_total = nb * 128
    nbw = nb // _NW
    nchunks = nbw // _CB

    @functools.partial(
        pl.kernel,
        mesh=mesh,
        out_type=jax.ShapeDtypeStruct((fout, e_total), jnp.float32),
        compiler_params=pltpu.CompilerParams(
            needs_layout_passes=False, use_tc_tiling_on_sc=False),
        scratch_types=[
            pltpu.VMEM((_CB, 128), jnp.int32),
            pltpu.VMEM((_CB * 128, 16), jnp.float32),
            pltpu.VMEM((16 * _CB * 128,), jnp.float32),
            pltpu.SemaphoreType.DMA,
        ],
    )
    def gather_k(table_hbm, idx_hbm, out_hbm, idx_v, rec_v, f1_v, sem):
        wid = lax.axis_index("s") * 2 + lax.axis_index("c")
        start = wid * nbw
        cw = _CB * 128
        iota = lax.iota(jnp.int32, 16)
        fbase = iota * cw

        def chunk_body(c, carry):
            bstart = start + c * _CB
            pltpu.sync_copy(idx_hbm.at[pl.ds(bstart, _CB)], idx_v)
            cps = []
            for i in range(_CB):
                cps.append(pltpu.async_copy(
                    table_hbm.at[idx_v.at[i]],
                    rec_v.at[pl.ds(i * 128, 128), :], sem))
            for cp in cps:
                cp.wait()

            def rec_body(b, carry2):
                for j in range(16):
                    r = b * 16 + j
                    vec = rec_v[r, :]
                    plsc.store_scatter(f1_v, [fbase + r], vec)
                return carry2

            lax.fori_loop(0, _CB * 8, rec_body, 0)
            ops = []
            for f in range(fout):
                ops.append(pltpu.async_copy(
                    f1_v.at[pl.ds(f * cw, cw)],
                    out_hbm.at[f, pl.ds(bstart * 128, cw)], sem))
            for op in ops:
                op.wait()
            return carry

        lax.fori_loop(0, nchunks, chunk_body, 0)

    return gather_k


def _mlp2(l0, l1, x):
    y = jax.nn.relu(x @ l0["W"].T + l0["b"])
    return y @ l1["W"].T + l1["b"]


def _pick_block(e):
    for cand in (2560, 1280, 640, 512, 256, 128, 64, 32, 16, 8):
        if e % cand == 0:
            return cand
    return e


def _edge_mlp_kernel(w1u_ref, w1p_ref, b1_ref, w2_ref, b2_ref, u_ref, pd_ref,
                     o_ref):
    z = jnp.dot(w1u_ref[...], u_ref[...], preferred_element_type=jnp.float32)
    z = z + jnp.dot(w1p_ref[...], pd_ref[...],
                    preferred_element_type=jnp.float32)
    z = jax.nn.relu(z + b1_ref[...])
    o_ref[...] = (jnp.dot(w2_ref[...], z, preferred_element_type=jnp.float32)
                  + b2_ref[...])


@functools.partial(jax.jit, static_argnames=())
def _edge_mlp(w1u, w1p, b1, w2, b2, u_t, pd_t):
    e = u_t.shape[1]
    eb = _pick_block(e)
    grid = (e // eb,)
    full = lambda i: (0, 0)
    blk = lambda i: (0, i)
    return pl.pallas_call(
        _edge_mlp_kernel,
        grid=grid,
        in_specs=[
            pl.BlockSpec(w1u.shape, full),
            pl.BlockSpec(w1p.shape, full),
            pl.BlockSpec(b1.shape, full),
            pl.BlockSpec(w2.shape, full),
            pl.BlockSpec(b2.shape, full),
            pl.BlockSpec((u_t.shape[0], eb), blk),
            pl.BlockSpec((pd_t.shape[0], eb), blk),
        ],
        out_specs=pl.BlockSpec((3, eb), blk),
        out_shape=jax.ShapeDtypeStruct((3, e), jnp.float32),
    )(w1u, w1p, b1, w2, b2, u_t, pd_t)


@functools.lru_cache(maxsize=None)
def _cached_gather(n_nodes, nb, fout):
    return _make_sc_gather(n_nodes, nb, fout)


_NH = 25088
_SCW = 2048


def _make_sc_scatter_max(e_total):
    mesh = plsc.VectorSubcoreMesh(core_axis_name="c", subcore_axis_name="s")
    ew = e_total // _NW
    nchunks = ew // _SCW
    tsz = _NH * 3 + 16

    @functools.partial(
        pl.kernel,
        mesh=mesh,
        out_type=jax.ShapeDtypeStruct((_NW, 2, _NH * 3), jnp.float32),
        compiler_params=pltpu.CompilerParams(
            needs_layout_passes=False, use_tc_tiling_on_sc=False),
        scratch_types=[
            pltpu.VMEM((tsz,), jnp.float32),
            pltpu.VMEM((3 * _SCW,), jnp.float32),
            pltpu.VMEM((_SCW,), jnp.int32),
            pltpu.VMEM((16,), jnp.int32),
            pltpu.VMEM((16,), jnp.float32),
            pltpu.SemaphoreType.DMA,
        ],
    )
    def scatter_k(msg_hbm, dst_hbm, ninf_hbm, out_hbm,
                  tab_v, m_v, d_v, s16, c16, sem):
        wid = lax.axis_index("s") * 2 + lax.axis_index("c")
        ebase = wid * ew
        iota = lax.iota(jnp.int32, 16)
        ninf = jnp.full((16,), -jnp.inf, jnp.float32)
        trash = _NH * 3 + iota
        idx_m = [jnp.maximum(iota - k, 0) for k in (1, 2, 4, 8)]
        idx_p1 = jnp.minimum(iota + 1, 15)

        for p in range(2):
            base = p * _NH
            pltpu.sync_copy(ninf_hbm, tab_v)

            def chunk_body(ch, carry):
                estart = ebase + ch * _SCW
                pltpu.sync_copy(dst_hbm.at[pl.ds(estart, _SCW)], d_v)
                cps = [pltpu.async_copy(
                    msg_hbm.at[c, pl.ds(estart, _SCW)],
                    m_v.at[pl.ds(c * _SCW, _SCW)], sem) for c in range(3)]
                for cp in cps:
                    cp.wait()

                def vreg_body(b, carry2):
                    dvec = d_v[pl.ds(b * 16, 16)]
                    lvec = dvec - base
                    cnt, _ = plsc.scan_count(dvec)
                    has_dup = jnp.max(cnt) > 1

                    def fast(_):
                        valid = (lvec >= 0) & (lvec < _NH)
                        cl3 = jnp.clip(lvec, 0, _NH - 1) * 3
                        for c in range(3):
                            val = m_v[pl.ds(b * 16 + c * _SCW, 16)]
                            cur = plsc.load_gather(tab_v, [cl3 + c])
                            comb = jnp.maximum(
                                val, jnp.where(valid, cur, ninf))
                            tidx = jnp.where(valid, cl3 + c, trash)
                            plsc.store_scatter(tab_v, [tidx], comb)
                        return 0

                    def slow(_):
                        skey, perm = plsc.sort_key_val(lvec, iota)
                        s16[...] = skey
                        svalid = (skey >= 0) & (skey < _NH)
                        masks = []
                        for k, im in zip((1, 2, 4, 8), idx_m):
                            sh = plsc.load_gather(s16, [im])
                            masks.append((skey == sh) & (iota >= k))
                        shp1 = plsc.load_gather(s16, [idx_p1])
                        lastm = (skey != shp1) | (iota == 15)
                        storem = lastm & svalid
                        cl3 = jnp.clip(skey, 0, _NH - 1) * 3
                        for c in range(3):
                            val = plsc.load_gather(
                                m_v, [c * _SCW + b * 16 + perm])
                            cur = plsc.load_gather(tab_v, [cl3 + c])
                            comb = jnp.maximum(
                                val, jnp.where(svalid, cur, ninf))
                            for mk, im in zip(masks, idx_m):
                                c16[...] = comb
                                shc = plsc.load_gather(c16, [im])
                                comb = jnp.where(
                                    mk, jnp.maximum(comb, shc), comb)
                            tidx = jnp.where(storem, cl3 + c, trash)
                            plsc.store_scatter(tab_v, [tidx], comb)
                        return 0

                    lax.cond(has_dup, slow, fast, 0)
                    return carry2

                lax.fori_loop(0, _SCW // 16, vreg_body, 0)
                return carry

            lax.fori_loop(0, nchunks, chunk_body, 0)
            pltpu.sync_copy(tab_v.at[pl.ds(0, _NH * 3)], out_hbm.at[wid, p])

    return scatter_k


@functools.lru_cache(maxsize=None)
def _cached_scatter(e_total):
    return _make_sc_scatter_max(e_total)


def kernel(x, params, edge_index, batch):
    n = x.shape[0]
    g = 64
    src = edge_index[0]
    dst = edge_index[1]
    e = src.shape[0]
    unit = 128 * _NW * _CB
    e_pad = -(-e // unit) * unit
    nb = e_pad // 128
    pos = x
    src_p = jnp.concatenate([src, jnp.full((e_pad - e,), n, jnp.int32)])
    dst_p = jnp.concatenate([dst, jnp.full((e_pad - e,), n, jnp.int32)])
    src2d = src_p.reshape(nb, 128)
    dst2d = dst_p.reshape(nb, 128)
    zrow = jnp.zeros((1, 16), jnp.float32)
    zpad = jnp.zeros((n, 10), jnp.float32)
    pos16 = jnp.concatenate(
        [jnp.concatenate([pos, jnp.zeros((n, 13), jnp.float32)], axis=1),
         zrow], axis=0)
    pd_t = _cached_gather(n + 1, nb, 3)(pos16, dst2d)

    h = x
    for ci in range(3):
        p = params["conv%d" % (ci + 1)]
        hn = _mlp2(p["h"][0], p["h"][1], h)
        tab16 = jnp.concatenate(
            [jnp.concatenate([hn, pos, zpad], axis=1), zrow], axis=0)
        u_t = _cached_gather(n + 1, nb, 6)(tab16, src2d)

        wf1 = p["f"][0]["W"]
        b1 = p["f"][0]["b"][:, None]
        w1u = wf1
        w1p = -wf1[:, 3:]
        w2 = p["f"][1]["W"]
        b2 = p["f"][1]["b"][:, None]

        msg_t = _edge_mlp(w1u, w1p, b1, w2, b2, u_t, pd_t)
        ninf_init = jnp.full((_NH * 3 + 16,), -jnp.inf, jnp.float32)
        part = _cached_scatter(e_pad)(msg_t, dst_p, ninf_init)
        agg = jnp.max(part, axis=0).reshape(2 * _NH, 3)[:n]
        agg = jnp.where(jnp.isfinite(agg), agg, 0.0)
        h = _mlp2(p["g"][0], p["g"][1], agg)
        if ci < 2:
            h = jax.nn.relu(h)

    onehot = (batch[None, :] == jnp.arange(g, dtype=batch.dtype)[:, None]
              ).astype(jnp.float32)
    counts = jnp.sum(onehot, axis=1)
    sums = onehot @ h
    pooled = sums / jnp.maximum(counts, 1.0)[:, None]
    logits = pooled @ params["linear"]["W"].T + params["linear"]["b"]
    return jax.nn.log_softmax(logits, axis=1)


# speedup vs baseline: 1.1937x; 1.0029x over previous
import functools

import jax
import jax.numpy as jnp
from jax import lax
from jax.experimental import pallas as pl
from jax.experimental.pallas import tpu as pltpu
from jax.experimental.pallas import tpu_sc as plsc

_NW = 32
_CB = 25


def _make_sc_gather(n_rows, nb, fout):
    mesh = plsc.VectorSubcoreMesh(core_axis_name="c", subcore_axis_name="s")
    e_total = nb * 128
    nbw = nb // _NW
    nchunks = nbw // _CB

    @functools.partial(
        pl.kernel,
        mesh=mesh,
        out_type=jax.ShapeDtypeStruct((fout, e_total), jnp.float32),
        compiler_params=pltpu.CompilerParams(
            needs_layout_passes=False, use_tc_tiling_on_sc=False),
        scratch_types=[
            pltpu.VMEM((_CB, 128), jnp.int32),
            pltpu.VMEM((_CB * 128, 16), jnp.float32),
            pltpu.VMEM((16 * _CB * 128,), jnp.float32),
            pltpu.SemaphoreType.DMA,
        ],
    )
    def gather_k(table_hbm, idx_hbm, out_hbm, idx_v, rec_v, f1_v, sem):
        wid = lax.axis_index("s") * 2 + lax.axis_index("c")
        start = wid * nbw
        cw = _CB * 128
        iota = lax.iota(jnp.int32, 16)
        fbase = iota * cw

        def chunk_body(c, carry):
            bstart = start + c * _CB
            pltpu.sync_copy(idx_hbm.at[pl.ds(bstart, _CB)], idx_v)
            cps = []
            for i in range(_CB):
                cps.append(pltpu.async_copy(
                    table_hbm.at[idx_v.at[i]],
                    rec_v.at[pl.ds(i * 128, 128), :], sem))
            for cp in cps:
                cp.wait()

            def rec_body(b, carry2):
                for j in range(16):
                    r = b * 16 + j
                    vec = rec_v[r, :]
                    plsc.store_scatter(f1_v, [fbase + r], vec)
                return carry2

            lax.fori_loop(0, _CB * 8, rec_body, 0)
            ops = []
            for f in range(fout):
                ops.append(pltpu.async_copy(
                    f1_v.at[pl.ds(f * cw, cw)],
                    out_hbm.at[f, pl.ds(bstart * 128, cw)], sem))
            for op in ops:
                op.wait()
            return carry

        lax.fori_loop(0, nchunks, chunk_body, 0)

    return gather_k


def _mlp2(l0, l1, x):
    y = jax.nn.relu(x @ l0["W"].T + l0["b"])
    return y @ l1["W"].T + l1["b"]


def _pick_block(e):
    for cand in (2560, 1280, 640, 512, 256, 128, 64, 32, 16, 8):
        if e % cand == 0:
            return cand
    return e


def _edge_mlp_kernel(w1u_ref, w1p_ref, b1_ref, w2_ref, b2_ref, u_ref, pd_ref,
                     o_ref):
    z = jnp.dot(w1u_ref[...], u_ref[...], preferred_element_type=jnp.float32)
    z = z + jnp.dot(w1p_ref[...], pd_ref[...],
                    preferred_element_type=jnp.float32)
    z = jax.nn.relu(z + b1_ref[...])
    o_ref[...] = (jnp.dot(w2_ref[...], z, preferred_element_type=jnp.float32)
                  + b2_ref[...])


@functools.partial(jax.jit, static_argnames=())
def _edge_mlp(w1u, w1p, b1, w2, b2, u_t, pd_t):
    e = u_t.shape[1]
    eb = _pick_block(e)
    grid = (e // eb,)
    full = lambda i: (0, 0)
    blk = lambda i: (0, i)
    return pl.pallas_call(
        _edge_mlp_kernel,
        grid=grid,
        in_specs=[
            pl.BlockSpec(w1u.shape, full),
            pl.BlockSpec(w1p.shape, full),
            pl.BlockSpec(b1.shape, full),
            pl.BlockSpec(w2.shape, full),
            pl.BlockSpec(b2.shape, full),
            pl.BlockSpec((u_t.shape[0], eb), blk),
            pl.BlockSpec((pd_t.shape[0], eb), blk),
        ],
        out_specs=pl.BlockSpec((3, eb), blk),
        out_shape=jax.ShapeDtypeStruct((3, e), jnp.float32),
    )(w1u, w1p, b1, w2, b2, u_t, pd_t)


@functools.lru_cache(maxsize=None)
def _cached_gather(n_nodes, nb, fout):
    return _make_sc_gather(n_nodes, nb, fout)


_NH = 25088
_SCW = 2048


def _make_sc_scatter_max(e_total):
    mesh = plsc.VectorSubcoreMesh(core_axis_name="c", subcore_axis_name="s")
    ew = e_total // _NW
    nchunks = ew // _SCW
    tsz = _NH * 3 + 16

    @functools.partial(
        pl.kernel,
        mesh=mesh,
        out_type=jax.ShapeDtypeStruct((_NW, 2, _NH * 3), jnp.float32),
        compiler_params=pltpu.CompilerParams(
            needs_layout_passes=False, use_tc_tiling_on_sc=False),
        scratch_types=[
            pltpu.VMEM((tsz,), jnp.float32),
            pltpu.VMEM((3 * _SCW,), jnp.float32),
            pltpu.VMEM((_SCW,), jnp.int32),
            pltpu.VMEM((16,), jnp.int32),
            pltpu.VMEM((16,), jnp.float32),
            pltpu.SemaphoreType.DMA,
        ],
    )
    def scatter_k(msg_hbm, dst_hbm, ninf_hbm, out_hbm,
                  tab_v, m_v, d_v, s16, c16, sem):
        wid = lax.axis_index("s") * 2 + lax.axis_index("c")
        ebase = wid * ew
        iota = lax.iota(jnp.int32, 16)
        ninf = jnp.full((16,), -jnp.inf, jnp.float32)
        trash = _NH * 3 + iota
        idx_m = [jnp.maximum(iota - k, 0) for k in (1, 2, 4, 8)]
        idx_p1 = jnp.minimum(iota + 1, 15)

        for p in range(2):
            base = p * _NH
            pltpu.sync_copy(ninf_hbm, tab_v)

            def chunk_body(ch, carry):
                estart = ebase + ch * _SCW
                pltpu.sync_copy(dst_hbm.at[pl.ds(estart, _SCW)], d_v)
                cps = [pltpu.async_copy(
                    msg_hbm.at[c, pl.ds(estart, _SCW)],
                    m_v.at[pl.ds(c * _SCW, _SCW)], sem) for c in range(3)]
                for cp in cps:
                    cp.wait()

                def vreg_body(b, carry2):
                    dvec = d_v[pl.ds(b * 16, 16)]
                    lvec = dvec - base
                    cnt, _ = plsc.scan_count(dvec)
                    has_dup = jnp.max(cnt) > 1

                    def fast(_):
                        valid = (lvec >= 0) & (lvec < _NH)
                        cl3 = jnp.clip(lvec, 0, _NH - 1) * 3
                        for c in range(3):
                            val = m_v[pl.ds(b * 16 + c * _SCW, 16)]
                            cur = plsc.load_gather(tab_v, [cl3 + c])
                            comb = jnp.maximum(
                                val, jnp.where(valid, cur, ninf))
                            tidx = jnp.where(valid, cl3 + c, trash)
                            plsc.store_scatter(tab_v, [tidx], comb)
                        return 0

                    def slow(_):
                        skey, perm = plsc.sort_key_val(lvec, iota)
                        s16[...] = skey
                        svalid = (skey >= 0) & (skey < _NH)
                        masks = []
                        for k, im in zip((1, 2, 4, 8), idx_m):
                            sh = plsc.load_gather(s16, [im])
                            masks.append((skey == sh) & (iota >= k))
                        shp1 = plsc.load_gather(s16, [idx_p1])
                        lastm = (skey != shp1) | (iota == 15)
                        storem = lastm & svalid
                        cl3 = jnp.clip(skey, 0, _NH - 1) * 3
                        for c in range(3):
                            val = plsc.load_gather(
                                m_v, [c * _SCW + b * 16 + perm])
                            cur = plsc.load_gather(tab_v, [cl3 + c])
                            comb = jnp.maximum(
                                val, jnp.where(svalid, cur, ninf))
                            for mk, im in zip(masks, idx_m):
                                c16[...] = comb
                                shc = plsc.load_gather(c16, [im])
                                comb = jnp.where(
                                    mk, jnp.maximum(comb, shc), comb)
                            tidx = jnp.where(storem, cl3 + c, trash)
                            plsc.store_scatter(tab_v, [tidx], comb)
                        return 0

                    lax.cond(has_dup, slow, fast, 0)
                    return carry2

                lax.fori_loop(0, _SCW // 16, vreg_body, 0)
                return carry

            lax.fori_loop(0, nchunks, chunk_body, 0)
            pltpu.sync_copy(tab_v.at[pl.ds(0, _NH * 3)], out_hbm.at[wid, p])

    return scatter_k


@functools.lru_cache(maxsize=None)
def _cached_scatter(e_total):
    return _make_sc_scatter_max(e_total)


def kernel(x, params, edge_index, batch):
    n = x.shape[0]
    g = 64
    src = edge_index[0]
    dst = edge_index[1]
    e = src.shape[0]
    unit = 128 * _NW * _CB
    e_pad = -(-e // unit) * unit
    nb = e_pad // 128
    pos = x
    src_p = jnp.concatenate([src, jnp.full((e_pad - e,), n, jnp.int32)])
    dst_p = jnp.concatenate([dst, jnp.full((e_pad - e,), n, jnp.int32)])
    src2d = src_p.reshape(nb, 128)
    dst2d = dst_p.reshape(nb, 128)
    zrow = jnp.zeros((1, 16), jnp.float32)
    zpad = jnp.zeros((n, 10), jnp.float32)
    pos16 = jnp.concatenate(
        [jnp.concatenate([pos, jnp.zeros((n, 13), jnp.float32)], axis=1),
         zrow], axis=0)
    pd_t = _cached_gather(n + 1, nb, 3)(pos16, dst2d)

    h = x
    for ci in range(3):
        p = params["conv%d" % (ci + 1)]
        hn = _mlp2(p["h"][0], p["h"][1], h)
        tab16 = jnp.concatenate(
            [jnp.concatenate([hn, pos, zpad], axis=1), zrow], axis=0)
        u_t = _cached_gather(n + 1, nb, 6)(tab16, src2d)

        wf1 = p["f"][0]["W"]
        b1 = p["f"][0]["b"][:, None]
        w1u = wf1
        w1p = -wf1[:, 3:]
        w2 = p["f"][1]["W"]
        b2 = p["f"][1]["b"][:, None]

        msg_t = _edge_mlp(w1u, w1p, b1, w2, b2, u_t, pd_t)
        ninf_init = jnp.full((_NH * 3 + 16,), -jnp.inf, jnp.float32)
        part = _cached_scatter(e_pad)(msg_t, dst_p, ninf_init)
        agg = jnp.max(part, axis=0).reshape(2 * _NH, 3)[:n]
        agg = jnp.where(jnp.isfinite(agg), agg, 0.0)
        h = _mlp2(p["g"][0], p["g"][1], agg)
        if ci < 2:
            h = jax.nn.relu(h)

    onehot = (batch[None, :] == jnp.arange(g, dtype=batch.dtype)[:, None]
              ).astype(jnp.float32)
    counts = jnp.sum(onehot, axis=1)
    sums = onehot @ h
    pooled = sums / jnp.maximum(counts, 1.0)[:, None]
    logits = pooled @ params["linear"]["W"].T + params["linear"]["b"]
    return jax.nn.log_softmax(logits, axis=1)
